# baseline (speedup 1.0000x reference)
import functools

import jax
import jax.numpy as jnp
from jax import lax
from jax.experimental import pallas as pl
from jax.experimental.pallas import tpu as pltpu
from jax.experimental.pallas import tpu_sc as plsc

_B, _P, _S, _H, _N = 64, 10, 20, 128, 128
_BP = _B * _P
_L = 16
_NW = 32
_CHUNKS = _BP // _L

_OFF_BM = 0
_OFF_BI = 128
_OFF_BH = 512
_OFF_NPX = 896
_OFF_NPY = 1024
_OFF_ST = 1152
_OFF_EN = 1216
_OFF_VA = 1280
_OFF_P2B = 1344
_PACK_LEN = 1984


def _sc_body(pack_hbm, lanes_hbm, mask_out, pack_v, ch_v, outm_v):
    wid = lax.axis_index("s") * 2 + lax.axis_index("c")
    pltpu.sync_copy(pack_hbm, pack_v)

    def _do_chunk(c):
        pltpu.sync_copy(lanes_hbm.at[c], ch_v)
        pair = lax.broadcasted_iota(jnp.int32, (_L,), 0) + c * _L
        b_ids = plsc.load_gather(pack_v, [pair + _OFF_P2B]).astype(jnp.int32)
        sidx_f = plsc.load_gather(pack_v, [b_ids + _OFF_ST])
        eidx_f = plsc.load_gather(pack_v, [b_ids + _OFF_EN])
        vld_f = plsc.load_gather(pack_v, [b_ids + _OFF_VA])
        sidx = sidx_f.astype(jnp.int32)
        npx = plsc.load_gather(pack_v, [sidx + _OFF_NPX])
        npy = plsc.load_gather(pack_v, [sidx + _OFF_NPY])
        cond = jnp.logical_and(vld_f > 0.0, (eidx_f - sidx_f) > 0.0)
        d2min = jnp.full((_L,), 3.4e38, jnp.float32)
        nan_any = jnp.zeros((_L,), jnp.bool_)
        for s in range(_S):
            lxs = ch_v[s]
            lys = ch_v[_S + s]
            nan_any = nan_any | (lxs != lxs) | (lys != lys)
            dx = npx - lxs
            dy = npy - lys
            d2min = jnp.minimum(d2min, dx * dx + dy * dy)
        d2 = jnp.where(nan_any, npx * npx + npy * npy, d2min)
        keep = jnp.logical_and(cond, d2 < 10000.0)
        outm_v[0] = jnp.where(keep, 1.0, 0.0).astype(jnp.float32)
        pltpu.sync_copy(outm_v, mask_out.at[c])

    _do_chunk(wid)

    @pl.when(wid < _CHUNKS - _NW)
    def _second_round():
        _do_chunk(wid + _NW)


_sc_kernel = functools.partial(
    pl.kernel,
    out_type=jax.ShapeDtypeStruct((_CHUNKS, 1, _L), jnp.float32),
    mesh=plsc.VectorSubcoreMesh(core_axis_name="c", subcore_axis_name="s"),
    compiler_params=pltpu.CompilerParams(use_tc_tiling_on_sc=False,
                                         needs_layout_passes=False),
    scratch_types=[
        pltpu.VMEM((_PACK_LEN,), jnp.float32),
        pltpu.VMEM((2 * _S, _L), jnp.float32),
        pltpu.VMEM((1, _L), jnp.float32),
    ],
)(_sc_body)


def _dn(a, b):
    return jax.lax.dot_general(a, b, (((1,), (1,)), ((), ())),
                               preferred_element_type=jnp.float32)


def _tc_body(B, P, S, H, N,
             pack_ref, actx_ref, nctx_tab_ref, mask_ref,
             Wm_ref, Wih_ref, Whh_ref, out_ref):
    BP = B * P
    pack = pack_ref[...]
    bm = pack[_OFF_BM:_OFF_BM + H]
    bi = pack[_OFF_BI:_OFF_BI + 3 * H]
    bh = pack[_OFF_BH:_OFF_BH + 3 * H]
    npx_tab = pack[_OFF_NPX:_OFF_NPX + N].reshape(1, N)
    npy_tab = pack[_OFF_NPY:_OFF_NPY + N].reshape(1, N)
    starts = pack[_OFF_ST:_OFF_ST + B].reshape(1, B)

    iota_n = jax.lax.broadcasted_iota(jnp.int32, (N, B), 0).astype(jnp.float32)
    onehotT = (iota_n == starts).astype(jnp.float32)

    nctx = jax.lax.dot_general(onehotT, nctx_tab_ref[...],
                               (((0,), (0,)), ((), ())),
                               preferred_element_type=jnp.float32)
    npx_row = jnp.dot(npx_tab, onehotT, preferred_element_type=jnp.float32)
    npy_row = jnp.dot(npy_tab, onehotT, preferred_element_type=jnp.float32)
    npxy = jnp.concatenate([jnp.transpose(npx_row), jnp.transpose(npy_row)],
                           axis=1)

    actx = actx_ref[...]
    Wm = Wm_ref[...]
    xg = (_dn(nctx, Wm[:, 2:2 + H]) + _dn(actx, Wm[:, 2 + H:])
          + _dn(-npxy, Wm[:, 0:2]) + bm)
    x = jnp.maximum(xg, 0.0)

    gi = _dn(x, Wih_ref[...]) + bi
    gh = _dn(nctx, Whh_ref[...]) + bh
    r_g = jax.nn.sigmoid(gi[:, :H] + gh[:, :H])
    z = jax.nn.sigmoid(gi[:, H:2 * H] + gh[:, H:2 * H])
    n_g = jnp.tanh(gi[:, 2 * H:] + r_g * gh[:, 2 * H:])
    r = (1.0 - z) * n_g + z * nctx

    iota_p = jax.lax.broadcasted_iota(jnp.int32, (BP, B), 0)
    iota_b = jax.lax.broadcasted_iota(jnp.int32, (BP, B), 1)
    rep = jnp.logical_and(iota_p >= P * iota_b,
                          iota_p < P * iota_b + P).astype(jnp.float32)
    r_rep = jnp.dot(rep, r, preferred_element_type=jnp.float32)

    keep = mask_ref[...].reshape(BP, 1) > 0.5
    out_ref[...] = jnp.where(keep, r_rep, 0.0).reshape(B, P, H)


def kernel(agent_pos, agent_context, ngh_pos, ngh_context, possible_lanes,
           lane_context, label, seq_start_end, valid_neighbor,
           W_msg, b_msg, W_ih, W_hh, b_ih, b_hh):
    B, P, H = lane_context.shape
    S = possible_lanes.shape[0]
    N = ngh_context.shape[0]

    pack = jnp.concatenate([
        b_msg, b_ih, b_hh,
        ngh_pos[:, 0], ngh_pos[:, 1],
        seq_start_end[:, 0].astype(jnp.float32),
        seq_start_end[:, 1].astype(jnp.float32),
        valid_neighbor.astype(jnp.float32),
        (jnp.arange(B * P, dtype=jnp.int32) // P).astype(jnp.float32),
    ])
    lxc = possible_lanes[:, :, 0].reshape(S, _CHUNKS, _L).transpose(1, 0, 2)
    lyc = possible_lanes[:, :, 1].reshape(S, _CHUNKS, _L).transpose(1, 0, 2)
    lanes_sc = jnp.concatenate([lxc, lyc], axis=1)

    mask3 = _sc_kernel(pack, lanes_sc)

    body = functools.partial(_tc_body, B, P, S, H, N)
    out2 = pl.pallas_call(
        body,
        out_shape=jax.ShapeDtypeStruct((B, P, H), jnp.float32),
    )(pack, agent_context, ngh_context, mask3.reshape(B * P), W_msg,
      W_ih, W_hh)

    return (lane_context, out2)


# speedup vs baseline: 1.0167x; 1.0167x over previous
import functools

import jax
import jax.numpy as jnp
from jax import lax
from jax.experimental import pallas as pl
from jax.experimental.pallas import tpu as pltpu
from jax.experimental.pallas import tpu_sc as plsc

_B, _P, _S, _H, _N = 64, 10, 20, 128, 128
_BP = _B * _P
_L = 16
_NW = 32
_CHUNKS = _BP // _L

_OFF_BM = 0
_OFF_BI = 128
_OFF_BH = 512
_OFF_NPX = 896
_OFF_NPY = 1024
_OFF_ST = 1152
_OFF_EN = 1216
_OFF_VA = 1280
_OFF_P2B = 1344
_PACK_LEN = 1984
_SC_LEN = _PACK_LEN - _OFF_NPX


def _sc_body(pack_hbm, lanes_hbm, mask_out, pack_v, ch_v, outm_v,
             sem_p, sem_l):
    wid = lax.axis_index("s") * 2 + lax.axis_index("c")
    cp_p = pltpu.async_copy(pack_hbm.at[pl.ds(_OFF_NPX, _SC_LEN)],
                            pack_v, sem_p)
    cp_l = pltpu.async_copy(lanes_hbm.at[wid], ch_v, sem_l)
    cp_p.wait()
    cp_l.wait()

    def _do_chunk(c, first):
        if not first:
            pltpu.sync_copy(lanes_hbm.at[c], ch_v)
        pair = lax.broadcasted_iota(jnp.int32, (_L,), 0) + c * _L
        b_ids = plsc.load_gather(
            pack_v, [pair + (_OFF_P2B - _OFF_NPX)]).astype(jnp.int32)
        sidx_f = plsc.load_gather(pack_v, [b_ids + (_OFF_ST - _OFF_NPX)])
        eidx_f = plsc.load_gather(pack_v, [b_ids + (_OFF_EN - _OFF_NPX)])
        vld_f = plsc.load_gather(pack_v, [b_ids + (_OFF_VA - _OFF_NPX)])
        sidx = sidx_f.astype(jnp.int32)
        npx = plsc.load_gather(pack_v, [sidx])
        npy = plsc.load_gather(pack_v, [sidx + (_OFF_NPY - _OFF_NPX)])
        cond = jnp.logical_and(vld_f > 0.0, (eidx_f - sidx_f) > 0.0)
        d2min = jnp.full((_L,), 3.4e38, jnp.float32)
        nan_any = jnp.zeros((_L,), jnp.bool_)
        for s in range(_S):
            lxs = ch_v[s]
            lys = ch_v[_S + s]
            nan_any = nan_any | (lxs != lxs) | (lys != lys)
            dx = npx - lxs
            dy = npy - lys
            d2min = jnp.minimum(d2min, dx * dx + dy * dy)
        d2 = jnp.where(nan_any, npx * npx + npy * npy, d2min)
        keep = jnp.logical_and(cond, d2 < 10000.0)
        outm_v[0] = jnp.where(keep, 1.0, 0.0).astype(jnp.float32)
        pltpu.sync_copy(outm_v, mask_out.at[c])

    _do_chunk(wid, True)

    @pl.when(wid < _CHUNKS - _NW)
    def _second_round():
        _do_chunk(wid + _NW, False)


_sc_kernel = functools.partial(
    pl.kernel,
    out_type=jax.ShapeDtypeStruct((_CHUNKS, 1, _L), jnp.float32),
    mesh=plsc.VectorSubcoreMesh(core_axis_name="c", subcore_axis_name="s"),
    compiler_params=pltpu.CompilerParams(use_tc_tiling_on_sc=False,
                                         needs_layout_passes=False),
    scratch_types=[
        pltpu.VMEM((_SC_LEN,), jnp.float32),
        pltpu.VMEM((2 * _S, _L), jnp.float32),
        pltpu.VMEM((1, _L), jnp.float32),
        pltpu.SemaphoreType.DMA,
        pltpu.SemaphoreType.DMA,
    ],
)(_sc_body)


def _dn(a, b):
    return jax.lax.dot_general(a, b, (((1,), (1,)), ((), ())),
                               preferred_element_type=jnp.float32)


def _tc_body(B, P, S, H, N,
             pack_ref, actx_ref, nctx_tab_ref, mask_ref,
             Wm_ref, Wih_ref, Whh_ref, out_ref):
    BP = B * P
    pack = pack_ref[...]
    bm = pack[_OFF_BM:_OFF_BM + H]
    bi = pack[_OFF_BI:_OFF_BI + 3 * H]
    bh = pack[_OFF_BH:_OFF_BH + 3 * H]
    npx_tab = pack[_OFF_NPX:_OFF_NPX + N].reshape(1, N)
    npy_tab = pack[_OFF_NPY:_OFF_NPY + N].reshape(1, N)
    starts = pack[_OFF_ST:_OFF_ST + B].reshape(1, B)

    iota_n = jax.lax.broadcasted_iota(jnp.int32, (N, B), 0).astype(jnp.float32)
    onehotT = (iota_n == starts).astype(jnp.float32)

    nctx = jax.lax.dot_general(onehotT, nctx_tab_ref[...],
                               (((0,), (0,)), ((), ())),
                               preferred_element_type=jnp.float32)
    npx_row = jnp.dot(npx_tab, onehotT, preferred_element_type=jnp.float32)
    npy_row = jnp.dot(npy_tab, onehotT, preferred_element_type=jnp.float32)
    npxy = jnp.concatenate([jnp.transpose(npx_row), jnp.transpose(npy_row)],
                           axis=1)

    actx = actx_ref[...]
    Wm = Wm_ref[...]
    xg = (_dn(nctx, Wm[:, 2:2 + H]) + _dn(actx, Wm[:, 2 + H:])
          + _dn(-npxy, Wm[:, 0:2]) + bm)
    x = jnp.maximum(xg, 0.0)

    gi = _dn(x, Wih_ref[...]) + bi
    gh = _dn(nctx, Whh_ref[...]) + bh
    r_g = jax.nn.sigmoid(gi[:, :H] + gh[:, :H])
    z = jax.nn.sigmoid(gi[:, H:2 * H] + gh[:, H:2 * H])
    n_g = jnp.tanh(gi[:, 2 * H:] + r_g * gh[:, 2 * H:])
    r = (1.0 - z) * n_g + z * nctx

    iota_p = jax.lax.broadcasted_iota(jnp.int32, (BP, B), 0)
    iota_b = jax.lax.broadcasted_iota(jnp.int32, (BP, B), 1)
    rep = jnp.logical_and(iota_p >= P * iota_b,
                          iota_p < P * iota_b + P).astype(jnp.float32)
    r_rep = jnp.dot(rep, r, preferred_element_type=jnp.float32)

    keep = mask_ref[...].reshape(BP, 1) > 0.5
    out_ref[...] = jnp.where(keep, r_rep, 0.0).reshape(B, P, H)


def kernel(agent_pos, agent_context, ngh_pos, ngh_context, possible_lanes,
           lane_context, label, seq_start_end, valid_neighbor,
           W_msg, b_msg, W_ih, W_hh, b_ih, b_hh):
    B, P, H = lane_context.shape
    S = possible_lanes.shape[0]
    N = ngh_context.shape[0]

    pack = jnp.concatenate([
        b_msg, b_ih, b_hh,
        ngh_pos[:, 0], ngh_pos[:, 1],
        seq_start_end[:, 0].astype(jnp.float32),
        seq_start_end[:, 1].astype(jnp.float32),
        valid_neighbor.astype(jnp.float32),
        (jnp.arange(B * P, dtype=jnp.int32) // P).astype(jnp.float32),
    ])
    lxc = possible_lanes[:, :, 0].reshape(S, _CHUNKS, _L).transpose(1, 0, 2)
    lyc = possible_lanes[:, :, 1].reshape(S, _CHUNKS, _L).transpose(1, 0, 2)
    lanes_sc = jnp.concatenate([lxc, lyc], axis=1)

    mask3 = _sc_kernel(pack, lanes_sc)

    body = functools.partial(_tc_body, B, P, S, H, N)
    out2 = pl.pallas_call(
        body,
        out_shape=jax.ShapeDtypeStruct((B, P, H), jnp.float32),
    )(pack, agent_context, ngh_context, mask3.reshape(B * P), W_msg,
      W_ih, W_hh)

    return (lane_context, out2)
